# Initial kernel scaffold; baseline (speedup 1.0000x reference)
import jax
import jax.numpy as jnp
from jax.experimental import pallas as pl


def kernel(entity_emb, items, labels, user_triple_set, item_triple_set):
    raise NotImplementedError("write your pallas kernel here")


# speedup vs baseline: 6.0869x; 6.0869x over previous
import functools

import jax
import jax.numpy as jnp
from jax import lax
from jax.experimental import pallas as pl
from jax.experimental.pallas import tpu as pltpu
from jax.experimental.pallas import tpu_sc as plsc

DIM = 128
M = 200
B = 4096
NC, NS = 2, 16
NW = NC * NS
EPW = B // NW
MA = 128
MB = M - MA


def _sc_embed(emb, uidx, items):
    mesh = plsc.VectorSubcoreMesh(core_axis_name="c", subcore_axis_name="s")
    out_type = (
        jax.ShapeDtypeStruct((B, DIM), jnp.float32),
        jax.ShapeDtypeStruct((B, DIM), jnp.float32),
    )

    @functools.partial(
        pl.kernel,
        mesh=mesh,
        out_type=out_type,
        scratch_types=[
            pltpu.VMEM((EPW * M,), jnp.int32),
            pltpu.VMEM((MA, DIM), jnp.float32),
            pltpu.VMEM((MB, DIM), jnp.float32),
            pltpu.VMEM((EPW, DIM), jnp.float32),
            pltpu.VMEM((EPW,), jnp.int32),
            pltpu.VMEM((EPW, DIM), jnp.float32),
            pltpu.SemaphoreType.DMA,
        ],
    )
    def k(emb_hbm, uidx_hbm, items_hbm, eu_hbm, ev_hbm,
          idx_v, rows_a, rows_b, eu_v, it_v, ev_v, sem):
        wid = lax.axis_index("s") * NC + lax.axis_index("c")
        base = wid * EPW

        pltpu.sync_copy(items_hbm.at[pl.ds(base, EPW)], it_v)
        pltpu.async_copy(emb_hbm.at[it_v], ev_v, sem).wait()
        pltpu.sync_copy(ev_v, ev_hbm.at[pl.ds(base, EPW)])

        pltpu.sync_copy(uidx_hbm.at[pl.ds(base * M, EPW * M)], idx_v)

        @pl.loop(0, EPW)
        def per_elem(e):
            off_a = pl.multiple_of(e * M, 8)
            off_b = pl.multiple_of(e * M + MA, 8)
            pltpu.async_copy(
                emb_hbm.at[idx_v.at[pl.ds(off_a, MA)]], rows_a, sem).wait()
            pltpu.async_copy(
                emb_hbm.at[idx_v.at[pl.ds(off_b, MB)]], rows_b, sem).wait()

            def body_a(r, accs):
                return tuple(accs[c] + rows_a[r, pl.ds(c * 16, 16)]
                             for c in range(8))

            accs = lax.fori_loop(
                0, MA, body_a,
                tuple(jnp.zeros((16,), jnp.float32) for _ in range(8)))

            def body_b(r, accs):
                return tuple(accs[c] + rows_b[r, pl.ds(c * 16, 16)]
                             for c in range(8))

            accs = lax.fori_loop(0, MB, body_b, accs)
            for c in range(8):
                eu_v[e, pl.ds(c * 16, 16)] = accs[c] * (1.0 / M)

        pltpu.sync_copy(eu_v, eu_hbm.at[pl.ds(base, EPW)])

    return k(emb, uidx, items)


def _tc_score_body(eu_ref, ev_ref, y_ref, s_ref, loss_ref):
    d = jnp.sum(eu_ref[...] * ev_ref[...], axis=1, keepdims=True)
    s = jax.nn.sigmoid(d)
    s_ref[...] = s
    y = y_ref[...]
    eps = 1e-12
    sc = jnp.clip(s, eps, 1.0 - eps)
    bl = y * jnp.log(sc) + (1.0 - y) * jnp.log(1.0 - sc)
    loss_ref[...] = -jnp.sum(bl, axis=(0, 1), keepdims=True) * (1.0 / B)


def _tc_score(eu, ev, y):
    return pl.pallas_call(
        _tc_score_body,
        out_shape=(
            jax.ShapeDtypeStruct((B, 1), jnp.float32),
            jax.ShapeDtypeStruct((1, 1), jnp.float32),
        ),
    )(eu, ev, y)


def kernel(entity_emb, items, labels, user_triple_set, item_triple_set):
    uidx = user_triple_set[0, 0].astype(jnp.int32).reshape(-1)
    it = items.astype(jnp.int32)
    eu, ev = _sc_embed(entity_emb, uidx, it)
    y = labels.astype(jnp.float32).reshape(B, 1)
    s, loss = _tc_score(eu, ev, y)
    return s.reshape(B), loss[0, 0]


# speedup vs baseline: 12.6376x; 2.0762x over previous
import functools

import jax
import jax.numpy as jnp
from jax import lax
from jax.experimental import pallas as pl
from jax.experimental.pallas import tpu as pltpu
from jax.experimental.pallas import tpu_sc as plsc

DIM = 128
M = 200
B = 4096
NC, NS = 2, 16
NW = NC * NS
EPW = B // NW
MA = 128
MB = M - MA


def _sc_embed(emb, uidx, items):
    mesh = plsc.VectorSubcoreMesh(core_axis_name="c", subcore_axis_name="s")
    out_type = (
        jax.ShapeDtypeStruct((B, DIM), jnp.float32),
        jax.ShapeDtypeStruct((B, DIM), jnp.float32),
    )

    @functools.partial(
        pl.kernel,
        mesh=mesh,
        out_type=out_type,
        scratch_types=[
            pltpu.VMEM((EPW * M,), jnp.int32),
            pltpu.VMEM((MA, DIM), jnp.float32),
            pltpu.VMEM((MB, DIM), jnp.float32),
            pltpu.VMEM((MA, DIM), jnp.float32),
            pltpu.VMEM((MB, DIM), jnp.float32),
            pltpu.VMEM((EPW, DIM), jnp.float32),
            pltpu.VMEM((EPW,), jnp.int32),
            pltpu.VMEM((EPW, DIM), jnp.float32),
            pltpu.SemaphoreType.DMA,
            pltpu.SemaphoreType.DMA,
            pltpu.SemaphoreType.DMA,
        ],
    )
    def k(emb_hbm, uidx_hbm, items_hbm, eu_hbm, ev_hbm,
          idx_v, rows_a0, rows_b0, rows_a1, rows_b1, eu_v, it_v, ev_v,
          sem0, sem1, semx):
        wid = lax.axis_index("s") * NC + lax.axis_index("c")
        base = wid * EPW

        pltpu.sync_copy(items_hbm.at[pl.ds(base, EPW)], it_v)
        pltpu.async_copy(emb_hbm.at[it_v], ev_v, semx).wait()
        pltpu.sync_copy(ev_v, ev_hbm.at[pl.ds(base, EPW)])

        pltpu.sync_copy(uidx_hbm.at[pl.ds(base * M, EPW * M)], idx_v)

        def issue(e, ra, rb, sem):
            off_a = pl.multiple_of(e * M, 8)
            off_b = pl.multiple_of(e * M + MA, 8)
            pltpu.async_copy(emb_hbm.at[idx_v.at[pl.ds(off_a, MA)]], ra, sem)
            pltpu.async_copy(emb_hbm.at[idx_v.at[pl.ds(off_b, MB)]], rb, sem)

        def wait_set(ra, rb, sem):
            pltpu.make_async_copy(
                emb_hbm.at[idx_v.at[pl.ds(0, MA)]], ra, sem).wait()
            pltpu.make_async_copy(
                emb_hbm.at[idx_v.at[pl.ds(0, MB)]], rb, sem).wait()

        def accum(e, ra, rb):
            def body_a(r, accs):
                return tuple(accs[c] + ra[r, pl.ds(c * 16, 16)]
                             for c in range(8))

            accs = lax.fori_loop(
                0, MA, body_a,
                tuple(jnp.zeros((16,), jnp.float32) for _ in range(8)))

            def body_b(r, accs):
                return tuple(accs[c] + rb[r, pl.ds(c * 16, 16)]
                             for c in range(8))

            accs = lax.fori_loop(0, MB, body_b, accs)
            for c in range(8):
                eu_v[e, pl.ds(c * 16, 16)] = accs[c] * (1.0 / M)

        issue(0, rows_a0, rows_b0, sem0)

        @pl.loop(0, EPW // 2)
        def per_pair(g):
            e0 = g * 2
            issue(e0 + 1, rows_a1, rows_b1, sem1)
            wait_set(rows_a0, rows_b0, sem0)
            accum(e0, rows_a0, rows_b0)

            @pl.when(e0 + 2 < EPW)
            def _():
                issue(e0 + 2, rows_a0, rows_b0, sem0)

            wait_set(rows_a1, rows_b1, sem1)
            accum(e0 + 1, rows_a1, rows_b1)

        pltpu.sync_copy(eu_v, eu_hbm.at[pl.ds(base, EPW)])

    return k(emb, uidx, items)


def _tc_score_body(eu_ref, ev_ref, y_ref, s_ref, loss_ref):
    d = jnp.sum(eu_ref[...] * ev_ref[...], axis=1, keepdims=True)
    s = jax.nn.sigmoid(d)
    s_ref[...] = s
    y = y_ref[...]
    eps = 1e-12
    sc = jnp.clip(s, eps, 1.0 - eps)
    bl = y * jnp.log(sc) + (1.0 - y) * jnp.log(1.0 - sc)
    loss_ref[...] = -jnp.sum(bl, axis=(0, 1), keepdims=True) * (1.0 / B)


def _tc_score(eu, ev, y):
    return pl.pallas_call(
        _tc_score_body,
        out_shape=(
            jax.ShapeDtypeStruct((B, 1), jnp.float32),
            jax.ShapeDtypeStruct((1, 1), jnp.float32),
        ),
    )(eu, ev, y)


def kernel(entity_emb, items, labels, user_triple_set, item_triple_set):
    uidx = user_triple_set[0, 0].astype(jnp.int32).reshape(-1)
    it = items.astype(jnp.int32)
    eu, ev = _sc_embed(entity_emb, uidx, it)
    y = labels.astype(jnp.float32).reshape(B, 1)
    s, loss = _tc_score(eu, ev, y)
    return s.reshape(B), loss[0, 0]


# speedup vs baseline: 12.6533x; 1.0012x over previous
import functools

import jax
import jax.numpy as jnp
from jax import lax
from jax.experimental import pallas as pl
from jax.experimental.pallas import tpu as pltpu
from jax.experimental.pallas import tpu_sc as plsc

DIM = 128
M = 200
B = 4096
NC, NS = 2, 16
NW = NC * NS
EPW = B // NW
MA = 128
MB = M - MA


def _sc_embed(emb, uidx, items):
    mesh = plsc.VectorSubcoreMesh(core_axis_name="c", subcore_axis_name="s")
    out_type = (
        jax.ShapeDtypeStruct((B, DIM), jnp.float32),
        jax.ShapeDtypeStruct((B, DIM), jnp.float32),
    )

    @functools.partial(
        pl.kernel,
        mesh=mesh,
        out_type=out_type,
        scratch_types=[
            pltpu.VMEM((EPW * M,), jnp.int32),
            pltpu.VMEM((MA, DIM), jnp.float32),
            pltpu.VMEM((MB, DIM), jnp.float32),
            pltpu.VMEM((MA, DIM), jnp.float32),
            pltpu.VMEM((MB, DIM), jnp.float32),
            pltpu.VMEM((EPW, DIM), jnp.float32),
            pltpu.VMEM((EPW,), jnp.int32),
            pltpu.VMEM((EPW, DIM), jnp.float32),
            pltpu.SemaphoreType.DMA,
            pltpu.SemaphoreType.DMA,
            pltpu.SemaphoreType.DMA,
        ],
    )
    def k(emb_hbm, uidx_hbm, items_hbm, eu_hbm, ev_hbm,
          idx_v, rows_a0, rows_b0, rows_a1, rows_b1, eu_v, it_v, ev_v,
          sem0, sem1, semx):
        wid = lax.axis_index("s") * NC + lax.axis_index("c")
        base = wid * EPW

        pltpu.sync_copy(items_hbm.at[pl.ds(base, EPW)], it_v)
        pltpu.async_copy(emb_hbm.at[it_v], ev_v, semx).wait()
        pltpu.sync_copy(ev_v, ev_hbm.at[pl.ds(base, EPW)])

        pltpu.sync_copy(uidx_hbm.at[pl.ds(base * M, EPW * M)], idx_v)

        def issue(e, ra, rb, sem):
            off_a = pl.multiple_of(e * M, 8)
            off_b = pl.multiple_of(e * M + MA, 8)
            pltpu.async_copy(emb_hbm.at[idx_v.at[pl.ds(off_a, MA)]], ra, sem)
            pltpu.async_copy(emb_hbm.at[idx_v.at[pl.ds(off_b, MB)]], rb, sem)

        def wait_set(ra, rb, sem):
            pltpu.make_async_copy(
                emb_hbm.at[idx_v.at[pl.ds(0, MA)]], ra, sem).wait()
            pltpu.make_async_copy(
                emb_hbm.at[idx_v.at[pl.ds(0, MB)]], rb, sem).wait()

        def accum(e, ra, rb):
            def body_a(r4, accs):
                a = accs
                for u in range(4):
                    a = tuple(a[c] + ra[r4 * 4 + u, pl.ds(c * 16, 16)]
                              for c in range(8))
                return a

            accs = lax.fori_loop(
                0, MA // 4, body_a,
                tuple(jnp.zeros((16,), jnp.float32) for _ in range(8)))

            def body_b(r4, accs):
                a = accs
                for u in range(4):
                    a = tuple(a[c] + rb[r4 * 4 + u, pl.ds(c * 16, 16)]
                              for c in range(8))
                return a

            accs = lax.fori_loop(0, MB // 4, body_b, accs)
            for c in range(8):
                eu_v[e, pl.ds(c * 16, 16)] = accs[c] * (1.0 / M)

        issue(0, rows_a0, rows_b0, sem0)

        @pl.loop(0, EPW // 2)
        def per_pair(g):
            e0 = g * 2
            issue(e0 + 1, rows_a1, rows_b1, sem1)
            wait_set(rows_a0, rows_b0, sem0)
            accum(e0, rows_a0, rows_b0)

            @pl.when(e0 + 2 < EPW)
            def _():
                issue(e0 + 2, rows_a0, rows_b0, sem0)

            wait_set(rows_a1, rows_b1, sem1)
            accum(e0 + 1, rows_a1, rows_b1)

        pltpu.sync_copy(eu_v, eu_hbm.at[pl.ds(base, EPW)])

    return k(emb, uidx, items)


def _tc_score_body(eu_ref, ev_ref, y_ref, s_ref, loss_ref):
    d = jnp.sum(eu_ref[...] * ev_ref[...], axis=1, keepdims=True)
    s = jax.nn.sigmoid(d)
    s_ref[...] = s
    y = y_ref[...]
    eps = 1e-12
    sc = jnp.clip(s, eps, 1.0 - eps)
    bl = y * jnp.log(sc) + (1.0 - y) * jnp.log(1.0 - sc)
    loss_ref[...] = -jnp.sum(bl, axis=(0, 1), keepdims=True) * (1.0 / B)


def _tc_score(eu, ev, y):
    return pl.pallas_call(
        _tc_score_body,
        out_shape=(
            jax.ShapeDtypeStruct((B, 1), jnp.float32),
            jax.ShapeDtypeStruct((1, 1), jnp.float32),
        ),
    )(eu, ev, y)


def kernel(entity_emb, items, labels, user_triple_set, item_triple_set):
    uidx = user_triple_set[0, 0].astype(jnp.int32).reshape(-1)
    it = items.astype(jnp.int32)
    eu, ev = _sc_embed(entity_emb, uidx, it)
    y = labels.astype(jnp.float32).reshape(B, 1)
    s, loss = _tc_score(eu, ev, y)
    return s.reshape(B), loss[0, 0]
